# Initial kernel scaffold; baseline (speedup 1.0000x reference)
import jax
import jax.numpy as jnp
from jax.experimental import pallas as pl


def kernel(x_user, x_project, edge_index_up, edge_attr_up, edge_index_pu, edge_attr_pu, lin_user_W, lin_user_b, lin_proj_W, lin_proj_b, c1_up_Wl, c1_up_bl, c1_up_Wr, c1_up_br, c1_up_We, c1_up_att, c1_up_bias, c1_pu_Wl, c1_pu_bl, c1_pu_Wr, c1_pu_br, c1_pu_We, c1_pu_att, c1_pu_bias, c2_up_Wl, c2_up_bl, c2_up_Wr, c2_up_br, c2_up_We, c2_up_att, c2_up_bias, c2_pu_Wl, c2_pu_bl, c2_pu_Wr, c2_pu_br, c2_pu_We, c2_pu_att, c2_pu_bias, out_W, out_b):
    raise NotImplementedError("write your pallas kernel here")


# speedup vs baseline: 1.0733x; 1.0733x over previous
import jax
import jax.numpy as jnp
from jax.experimental import pallas as pl

H = 2
C = 64
NU = 50000
NP_ = 50000
E = 250000
PN = ('Wl', 'bl', 'Wr', 'br', 'We', 'att', 'bias')


def _gatv2(x_src, x_dst, src, dst, edge_attr, p, num_dst):
    xl = (x_src @ p['Wl'] + p['bl']).reshape(-1, H, C)
    xr = (x_dst @ p['Wr'] + p['br']).reshape(-1, H, C)
    e = (edge_attr @ p['We']).reshape(-1, H, C)
    x_j = xl[src]
    z = jax.nn.leaky_relu(x_j + xr[dst] + e, 0.2)
    alpha = jnp.sum(z * p['att'], axis=-1)
    gmax = jnp.max(alpha)
    ex = jnp.exp(alpha - gmax)
    den = jax.ops.segment_sum(ex, dst, num_segments=num_dst)
    num = jax.ops.segment_sum(x_j * ex[:, :, None], dst, num_segments=num_dst)
    out = num / (den[:, :, None] + 1e-16)
    return jnp.mean(out, axis=1) + p['bias']


def _proj_kernel(x_ref, w_ref, b_ref, o_ref):
    o_ref[:, :] = x_ref[:, :] @ w_ref[:, :] + b_ref[0, 0]


def _final_proj(xp, out_W, out_b):
    n = xp.shape[0]
    blk = 2000
    return pl.pallas_call(
        _proj_kernel,
        grid=(n // blk,),
        in_specs=[
            pl.BlockSpec((blk, C), lambda i: (i, 0)),
            pl.BlockSpec((C, 1), lambda i: (0, 0)),
            pl.BlockSpec((1, 1), lambda i: (0, 0)),
        ],
        out_specs=pl.BlockSpec((blk, 1), lambda i: (i, 0)),
        out_shape=jax.ShapeDtypeStruct((n, 1), jnp.float32),
    )(xp, out_W, out_b.reshape(1, 1))


def kernel(x_user, x_project, edge_index_up, edge_attr_up, edge_index_pu,
           edge_attr_pu, lin_user_W, lin_user_b, lin_proj_W, lin_proj_b,
           c1_up_Wl, c1_up_bl, c1_up_Wr, c1_up_br, c1_up_We, c1_up_att, c1_up_bias,
           c1_pu_Wl, c1_pu_bl, c1_pu_Wr, c1_pu_br, c1_pu_We, c1_pu_att, c1_pu_bias,
           c2_up_Wl, c2_up_bl, c2_up_Wr, c2_up_br, c2_up_We, c2_up_att, c2_up_bias,
           c2_pu_Wl, c2_pu_bl, c2_pu_Wr, c2_pu_br, c2_pu_We, c2_pu_att, c2_pu_bias,
           out_W, out_b):
    kw = dict(locals())
    xu = x_user @ lin_user_W + lin_user_b
    xp = x_project @ lin_proj_W + lin_proj_b
    for layer in ('c1', 'c2'):
        p_up = {n: kw[layer + '_up_' + n] for n in PN}
        p_pu = {n: kw[layer + '_pu_' + n] for n in PN}
        new_p = _gatv2(xu, xp, edge_index_up[0], edge_index_up[1],
                       edge_attr_up, p_up, NP_)
        new_u = _gatv2(xp, xu, edge_index_pu[0], edge_index_pu[1],
                       edge_attr_pu, p_pu, NU)
        xp = jax.nn.relu(new_p)
        xu = jax.nn.relu(new_u)
    return _final_proj(xp, out_W, out_b).reshape(-1)


# speedup vs baseline: 8.0245x; 7.4767x over previous
import functools

import jax
import jax.numpy as jnp
from jax import lax
from jax.experimental import pallas as pl
from jax.experimental.pallas import tpu as pltpu
from jax.experimental.pallas import tpu_sc as plsc

H = 2
C = 64
N = 50000
E = 250000
DIN = 128
HC = H * C

NC = 2
NS = 16
L = 16
NW = NC * NS

NA = 50048
RPT = NA // NS
B = 128
EPT = 7936
NBLK = EPT // B
EP = NW * EPT
G = B // L

_mesh = plsc.VectorSubcoreMesh(
    core_axis_name="c", subcore_axis_name="s", num_cores=NC, num_subcores=NS)


def _f32(*shape):
    return jax.ShapeDtypeStruct(shape, jnp.float32)


def _pass1_body(xl, xr, src, dst, a0, a1, we0, we1, at0, at1,
                alpha_out, gmax_out,
                we0_v, we1_v, atc_v,
                srcb, dstb, a0b, a1b, xlg, xrg, al0b, al1b, mbuf,
                sem1, sem2):
    cid = lax.axis_index("c")
    sid = lax.axis_index("s")
    wid = sid * NC + cid
    base = wid * EPT

    pltpu.sync_copy(we0, we0_v)
    pltpu.sync_copy(we1, we1_v)
    pltpu.sync_copy(at0, atc_v.at[pl.ds(0, C)])
    pltpu.sync_copy(at1, atc_v.at[pl.ds(C, C)])

    iota = lax.broadcasted_iota(jnp.int32, (L,), 0)
    rows = [iota + g * L for g in range(G)]

    def blk_body(blk, m):
        off = base + blk * B
        pltpu.sync_copy(src.at[pl.ds(off, B)], srcb)
        pltpu.sync_copy(dst.at[pl.ds(off, B)], dstb)
        pltpu.sync_copy(a0.at[pl.ds(off, B)], a0b)
        pltpu.sync_copy(a1.at[pl.ds(off, B)], a1b)
        d1 = pltpu.async_copy(xl.at[srcb], xlg, sem1)
        d2 = pltpu.async_copy(xr.at[dstb], xrg, sem2)
        d1.wait()
        d2.wait()

        attr0 = [a0b[pl.ds(g * L, L)] for g in range(G)]
        attr1 = [a1b[pl.ds(g * L, L)] for g in range(G)]
        zero = jnp.zeros((L,), jnp.float32)

        def jblk_body(j, accs):
            c0 = j * L
            w0v = we0_v[pl.ds(c0, L)]
            w1v = we1_v[pl.ds(c0, L)]
            av = atc_v[pl.ds(c0, L)]
            for k in range(L):
                c = c0 + k
                w0 = w0v[k]
                w1 = w1v[k]
                a = av[k]
                colv = jnp.full((L,), c, jnp.int32)
                out = []
                for g in range(G):
                    xlc = plsc.load_gather(xlg, [rows[g], colv])
                    xrc = plsc.load_gather(xrg, [rows[g], colv])
                    t = xlc + xrc + attr0[g] * w0 + attr1[g] * w1
                    lk = jnp.maximum(t, 0.2 * t)
                    out.append(accs[g] + lk * a)
                accs = tuple(out)
            return accs

        acc0 = lax.fori_loop(0, C // L, jblk_body, (zero,) * G)
        acc1 = lax.fori_loop(C // L, HC // L, jblk_body, (zero,) * G)

        for g in range(G):
            eid = off + rows[g]
            valid = eid < E
            v0 = jnp.where(valid, acc0[g], -1e30)
            v1 = jnp.where(valid, acc1[g], -1e30)
            al0b[pl.ds(g * L, L)] = v0
            al1b[pl.ds(g * L, L)] = v1
            m = jnp.maximum(m, jnp.maximum(v0, v1))

        pltpu.sync_copy(al0b, alpha_out.at[0, pl.ds(off, B)])
        pltpu.sync_copy(al1b, alpha_out.at[1, pl.ds(off, B)])
        return m

    m = lax.fori_loop(0, NBLK, blk_body, jnp.full((L,), -1e30, jnp.float32))
    mbuf[pl.ds(0, L)] = m
    pltpu.sync_copy(mbuf, gmax_out.at[pl.ds(wid * L, L)])


@functools.partial(
    pl.kernel,
    out_type=(_f32(2, EP), _f32(NW * L)),
    mesh=_mesh,
    compiler_params=pltpu.CompilerParams(use_tc_tiling_on_sc=False, needs_layout_passes=False),
    scratch_types=[
        pltpu.VMEM((HC,), jnp.float32),
        pltpu.VMEM((HC,), jnp.float32),
        pltpu.VMEM((HC,), jnp.float32),
        pltpu.VMEM((B,), jnp.int32),
        pltpu.VMEM((B,), jnp.int32),
        pltpu.VMEM((B,), jnp.float32),
        pltpu.VMEM((B,), jnp.float32),
        pltpu.VMEM((B, HC), jnp.float32),
        pltpu.VMEM((B, HC), jnp.float32),
        pltpu.VMEM((B,), jnp.float32),
        pltpu.VMEM((B,), jnp.float32),
        pltpu.VMEM((L,), jnp.float32),
        pltpu.SemaphoreType.DMA,
        pltpu.SemaphoreType.DMA,
    ],
)
def _sc_pass1(*refs):
    _pass1_body(*refs)


@functools.partial(
    pl.kernel,
    out_type=(_f32(NC, 4, NA, 32), _f32(NC, 2 * NA)),
    mesh=_mesh,
    compiler_params=pltpu.CompilerParams(use_tc_tiling_on_sc=False, needs_layout_passes=False),
    scratch_types=[
        pltpu.VMEM_SHARED((NA, 32), jnp.float32),
        pltpu.VMEM_SHARED((2 * NA,), jnp.float32),
        pltpu.VMEM((136, 32), jnp.float32),
        pltpu.VMEM((368,), jnp.float32),
        pltpu.VMEM((NW * L,), jnp.float32),
        pltpu.VMEM((B,), jnp.int32),
        pltpu.VMEM((B,), jnp.int32),
        pltpu.VMEM((B,), jnp.int32),
        pltpu.VMEM((B,), jnp.int32),
        pltpu.VMEM((B,), jnp.float32),
        pltpu.VMEM((B,), jnp.float32),
        pltpu.VMEM((B, 32), jnp.float32),
        pltpu.VMEM((B, 32), jnp.float32),
        pltpu.SemaphoreType.DMA,
    ],
)
def _sc_pass2(xl4, src, dst, alpha, gmaxs, nump, denp,
              accum, den_sh, zbuf, zbufd, gl,
              srcb, dstb, idxb, didxb, alb, exb, gv, wv, sem1):
    cid_ax = lax.axis_index("c")
    sid = lax.axis_index("s")
    wid = sid * NC + cid_ax
    base = wid * EPT

    iota = lax.broadcasted_iota(jnp.int32, (L,), 0)
    rows = [iota + g * L for g in range(G)]
    zv = jnp.zeros((L,), jnp.float32)

    def zb_body(r, _):
        zbuf[r, pl.ds(0, L)] = zv
        zbuf[r, pl.ds(L, L)] = zv
        return 0
    lax.fori_loop(0, 136, zb_body, 0)

    def zbd_body(i, _):
        zbufd[pl.ds(i * L, L)] = zv
        return 0
    lax.fori_loop(0, 368 // L, zbd_body, 0)

    pltpu.sync_copy(gmaxs, gl)
    mx = gl[pl.ds(0, L)]
    for i in range(1, NW):
        mx = jnp.maximum(mx, gl[pl.ds(i * L, L)])
    gmax = jnp.max(mx)

    for cid in range(4):
        h = cid // 2
        q = cid % 2
        def zacc_body(z, _):
            pltpu.sync_copy(zbuf, accum.at[pl.ds(sid * RPT + z * 136, 136)])
            return 0
        lax.fori_loop(0, RPT // 136, zacc_body, 0)
        if cid == 0:
            def zden_body(z, _):
                pltpu.sync_copy(
                    zbufd,
                    den_sh.at[pl.ds(sid * (2 * NA // NS) + z * 368, 368)])
                return 0
            lax.fori_loop(0, (2 * NA // NS) // 368, zden_body, 0)
        plsc.subcore_barrier()

        def blk_body(blk, _):
            off = base + blk * B
            pltpu.sync_copy(src.at[pl.ds(off, B)], srcb)
            pltpu.sync_copy(dst.at[pl.ds(off, B)], dstb)
            pltpu.sync_copy(alpha.at[h, pl.ds(off, B)], alb)
            exgs = []
            for g in range(G):
                av = alb[pl.ds(g * L, L)]
                exg = jnp.exp(av - gmax)
                exb[pl.ds(g * L, L)] = exg
                exgs.append(exg)
                sv = srcb[pl.ds(g * L, L)]
                idxb[pl.ds(g * L, L)] = sv * 4 + cid
                if q == 0:
                    dv = dstb[pl.ds(g * L, L)]
                    didxb[pl.ds(g * L, L)] = dv + h * NA
            pltpu.async_copy(xl4.at[idxb], gv, sem1).wait()
            for g in range(G):
                for c in range(32):
                    colv = jnp.full((L,), c, jnp.int32)
                    col = plsc.load_gather(gv, [rows[g], colv])
                    plsc.store_scatter(wv, [rows[g], colv], col * exgs[g])
            pltpu.sync_copy(wv, accum.at[dstb], add=True)
            if q == 0:
                pltpu.sync_copy(exb, den_sh.at[didxb], add=True)
            return 0

        lax.fori_loop(0, NBLK, blk_body, 0)
        plsc.subcore_barrier()
        pltpu.sync_copy(accum.at[pl.ds(sid * RPT, RPT)],
                        nump.at[cid_ax, cid, pl.ds(sid * RPT, RPT)])
        plsc.subcore_barrier()

    pltpu.sync_copy(den_sh.at[pl.ds(sid * (2 * NA // NS), 2 * NA // NS)],
                    denp.at[cid_ax, pl.ds(sid * (2 * NA // NS), 2 * NA // NS)])


BLK = 3128


def _mm_kernel(x_ref, w_ref, b_ref, o_ref):
    o_ref[:, :] = (
        jnp.dot(x_ref[:, :], w_ref[:, :], preferred_element_type=jnp.float32)
        + b_ref[0, :])


def _mm(x, w, b):
    n, k = x.shape
    m = w.shape[1]
    return pl.pallas_call(
        _mm_kernel,
        grid=(n // BLK,),
        in_specs=[
            pl.BlockSpec((BLK, k), lambda i: (i, 0)),
            pl.BlockSpec((k, m), lambda i: (0, 0)),
            pl.BlockSpec((1, m), lambda i: (0, 0)),
        ],
        out_specs=pl.BlockSpec((BLK, m), lambda i: (i, 0)),
        out_shape=jax.ShapeDtypeStruct((n, m), jnp.float32),
    )(x, w, b.reshape(1, m))


def _epi_kernel(nump_ref, denp_ref, bias_ref, o_ref):
    outs = []
    for q in range(2):
        acc = None
        for h in range(2):
            cid = h * 2 + q
            num = nump_ref[0, cid] + nump_ref[1, cid]
            den = denp_ref[0, h, :, 0] + denp_ref[1, h, :, 0] + 1e-16
            o = num / den[:, None]
            acc = o if acc is None else acc + o
        outs.append(0.5 * acc + bias_ref[0, q * 32:(q + 1) * 32])
    o_ref[:, :] = jnp.maximum(jnp.concatenate(outs, axis=1), 0.0)


def _epilogue(nump, denp, bias):
    denp3 = denp.reshape(NC, 2, NA, 1)
    return pl.pallas_call(
        _epi_kernel,
        grid=(NA // BLK,),
        in_specs=[
            pl.BlockSpec((NC, 4, BLK, 32), lambda i: (0, 0, i, 0)),
            pl.BlockSpec((NC, 2, BLK, 1), lambda i: (0, 0, i, 0)),
            pl.BlockSpec((1, C), lambda i: (0, 0)),
        ],
        out_specs=pl.BlockSpec((BLK, C), lambda i: (i, 0)),
        out_shape=jax.ShapeDtypeStruct((NA, C), jnp.float32),
    )(nump, denp3, bias.reshape(1, C))


def _pad1(a, n, val=0):
    return jnp.pad(a, (0, n - a.shape[0]), constant_values=val)


def _gat_sc(x_src, x_dst, src, dst, a0, a1, p):
    xl = _mm(x_src, p['Wl'], p['bl'])
    xr = _mm(x_dst, p['Wr'], p['br'])
    alpha, gmaxs = _sc_pass1(
        xl, xr, src, dst, a0, a1,
        p['We'][0], p['We'][1], p['att'][0, 0], p['att'][0, 1])
    nump, denp = _sc_pass2(xl.reshape(4 * NA, 32), src, dst, alpha, gmaxs)
    return _epilogue(nump, denp, p['bias'])


PN = ('Wl', 'bl', 'Wr', 'br', 'We', 'att', 'bias')


def kernel(x_user, x_project, edge_index_up, edge_attr_up, edge_index_pu,
           edge_attr_pu, lin_user_W, lin_user_b, lin_proj_W, lin_proj_b,
           c1_up_Wl, c1_up_bl, c1_up_Wr, c1_up_br, c1_up_We, c1_up_att, c1_up_bias,
           c1_pu_Wl, c1_pu_bl, c1_pu_Wr, c1_pu_br, c1_pu_We, c1_pu_att, c1_pu_bias,
           c2_up_Wl, c2_up_bl, c2_up_Wr, c2_up_br, c2_up_We, c2_up_att, c2_up_bias,
           c2_pu_Wl, c2_pu_bl, c2_pu_Wr, c2_pu_br, c2_pu_We, c2_pu_att, c2_pu_bias,
           out_W, out_b):
    kw = dict(locals())

    src_up = _pad1(edge_index_up[0], EP)
    dst_up = _pad1(edge_index_up[1], EP)
    src_pu = _pad1(edge_index_pu[0], EP)
    dst_pu = _pad1(edge_index_pu[1], EP)
    a0_up = _pad1(edge_attr_up[:, 0], EP)
    a1_up = _pad1(edge_attr_up[:, 1], EP)
    a0_pu = _pad1(edge_attr_pu[:, 0], EP)
    a1_pu = _pad1(edge_attr_pu[:, 1], EP)

    xu_in = jnp.pad(x_user, ((0, NA - N), (0, 0)))
    xp_in = jnp.pad(x_project, ((0, NA - N), (0, 0)))

    xu = _mm(xu_in, lin_user_W, lin_user_b)
    xp = _mm(xp_in, lin_proj_W, lin_proj_b)

    for layer in ('c1', 'c2'):
        p_up = {n: kw[layer + '_up_' + n] for n in PN}
        p_pu = {n: kw[layer + '_pu_' + n] for n in PN}
        new_p = _gat_sc(xu, xp, src_up, dst_up, a0_up, a1_up, p_up)
        new_u = _gat_sc(xp, xu, src_pu, dst_pu, a0_pu, a1_pu, p_pu)
        xp = new_p
        xu = new_u

    out = _mm(xp, out_W, out_b)
    return out[:N, 0]
